# Initial kernel scaffold; baseline (speedup 1.0000x reference)
import functools

import jax
import jax.numpy as jnp
from jax import lax
from jax.experimental import pallas as pl
from jax.experimental.pallas import tpu as pltpu
from jax.experimental.pallas import tpu_sc as plsc

M = 100000
D = 128
B = 16384
C = 100
L = 16
NW = 32
RW = 3136
NCHUNK = B // L
LCHUNK = RW // L
CAP = RW + L

_NEG = jnp.int32(-2147483648)


def _body(bx_h, lg_h, by_h, bt_h, x_h, ln_h, byn_h, idx_h, tv_h,
          out_bx, out_by, out_bt, out_lg,
          idx_v, win_v, rows_v, src_v, xbuf, lbuf, bybuf, tbuf,
          sem_bulk, sem_g, sem_s):
    cid = lax.axis_index("c")
    sid = lax.axis_index("s")
    wid = cid * 16 + sid
    lo = wid * RW
    hi = jnp.minimum(lo + RW, M)
    cbase = jnp.minimum(lo, M - RW)

    cp_bx = pltpu.make_async_copy(
        bx_h.at[pl.ds(cbase, RW)], out_bx.at[pl.ds(cbase, RW)], sem_bulk)
    cp_lg = pltpu.make_async_copy(
        lg_h.at[pl.ds(cbase, RW)], out_lg.at[pl.ds(cbase, RW)], sem_bulk)
    cp_by = pltpu.make_async_copy(
        by_h.at[pl.ds(cbase, RW)], out_by.at[pl.ds(cbase, RW)], sem_bulk)
    cp_bt = pltpu.make_async_copy(
        bt_h.at[pl.ds(cbase, RW)], out_bt.at[pl.ds(cbase, RW)], sem_bulk)
    cp_bx.start(); cp_lg.start(); cp_by.start(); cp_bt.start()

    pltpu.sync_copy(idx_h, idx_v)
    pltpu.sync_copy(tv_h, tbuf)
    iota = lax.iota(jnp.int32, L)
    neg1 = jnp.full((L,), -1, jnp.int32)

    def init_b(k, carry):
        win_v[pl.ds(k * L, L)] = neg1
        return carry
    lax.fori_loop(0, LCHUNK, init_b, 0)

    def scan_b(k, carry):
        v = idx_v[pl.ds(k * L, L)]
        m = (v >= lo) & (v < hi)
        cnt = jnp.sum(m.astype(jnp.int32))
        local = jnp.where(m, v - lo, 0)
        gi = k * L + iota

        @pl.when(cnt == 1)
        def _():
            plsc.store_scatter(win_v, [local], gi, mask=m)

        @pl.when(cnt > 1)
        def _():
            for j in range(L):
                plsc.store_scatter(win_v, [local], gi, mask=m & (iota == j))
        return carry
    lax.fori_loop(0, NCHUNK, scan_b, 0)

    def comp_b(k, off):
        w = win_v[pl.ds(k * L, L)]
        mv = w >= 0
        plsc.store_compressed(rows_v.at[pl.ds(off, L)], lo + k * L + iota,
                              mask=mv)
        plsc.store_compressed(src_v.at[pl.ds(off, L)], w, mask=mv)
        return off + jnp.sum(mv.astype(jnp.int32))
    n_upd = lax.fori_loop(0, LCHUNK, comp_b, jnp.int32(0))

    cp_bx.wait(); cp_lg.wait(); cp_by.wait(); cp_bt.wait()
    plsc.subcore_barrier()

    n_ch = (n_upd + (L - 1)) // L

    def upd_b(j, carry):
        rvec = rows_v[pl.ds(j * L, L)]
        svec = src_v[pl.ds(j * L, L)]
        valid = iota < (n_upd - j * L)
        r0 = jnp.max(jnp.where(iota == 0, rvec, _NEG))
        s0 = jnp.max(jnp.where(iota == 0, svec, _NEG))
        rvec = jnp.where(valid, rvec, r0)
        svec = jnp.where(valid, svec, s0)
        g_x = pltpu.make_async_copy(x_h.at[svec], xbuf, sem_g)
        g_l = pltpu.make_async_copy(ln_h.at[svec], lbuf, sem_g)
        g_b = pltpu.make_async_copy(byn_h.at[svec], bybuf, sem_g)
        g_x.start(); g_l.start(); g_b.start()
        g_x.wait(); g_l.wait(); g_b.wait()
        s_x = pltpu.make_async_copy(xbuf, out_bx.at[rvec], sem_s)
        s_l = pltpu.make_async_copy(lbuf, out_lg.at[rvec], sem_s)
        s_b = pltpu.make_async_copy(bybuf, out_by.at[rvec], sem_s)
        s_t = pltpu.make_async_copy(tbuf, out_bt.at[rvec], sem_s)
        s_x.start(); s_l.start(); s_b.start(); s_t.start()
        s_x.wait(); s_l.wait(); s_b.wait(); s_t.wait()
        return carry
    lax.fori_loop(0, n_ch, upd_b, 0)


@jax.jit
def _sc_scatter(bx, logits_buf, by_buf, bt_buf, x, logits_new, by_new,
                idx, tvec):
    f = pl.kernel(
        _body,
        out_type=(
            jax.ShapeDtypeStruct((M, D), jnp.float32),
            jax.ShapeDtypeStruct((M,), jnp.int32),
            jax.ShapeDtypeStruct((M,), jnp.int32),
            jax.ShapeDtypeStruct((M, C), jnp.float32),
        ),
        mesh=plsc.VectorSubcoreMesh(core_axis_name="c", subcore_axis_name="s"),
        scratch_types=[
            pltpu.VMEM((B,), jnp.int32),
            pltpu.VMEM((RW,), jnp.int32),
            pltpu.VMEM((CAP,), jnp.int32),
            pltpu.VMEM((CAP,), jnp.int32),
            pltpu.VMEM((L, D), jnp.float32),
            pltpu.VMEM((L, C), jnp.float32),
            pltpu.VMEM((L,), jnp.int32),
            pltpu.VMEM((L,), jnp.int32),
            pltpu.SemaphoreType.DMA,
            pltpu.SemaphoreType.DMA,
            pltpu.SemaphoreType.DMA,
        ],
    )
    return f(bx, logits_buf, by_buf, bt_buf, x, logits_new, by_new, idx, tvec)


def kernel(bx, logits_buf, by_buf, bt_buf, x, logits_new, by_new, idx, t):
    tvec = jnp.full((L,), t, dtype=jnp.int32)
    return _sc_scatter(bx, logits_buf, by_buf, bt_buf, x, logits_new,
                       by_new, idx.astype(jnp.int32), tvec)


# speedup vs baseline: 1.1049x; 1.1049x over previous
import jax
import jax.numpy as jnp
from jax import lax
from jax.experimental import pallas as pl
from jax.experimental.pallas import tpu as pltpu
from jax.experimental.pallas import tpu_sc as plsc

M = 100000
D = 128
B = 16384
C = 100
L = 16
NW = 32
RW = 3200
CH = 160
NCPC = RW // CH
NCHUNK = B // L
LVREG = CH // L
CAPC = CH + L


def _body(bx_h, lg_h, by_h, bt_h, x_h, lnp_h, byn_h, idx_h, tv_h,
          out_bx, out_by, out_bt, out_lg,
          idx_v, byn_v, win_v, loc_v, src_v, cpx, cpl, cby, cbt,
          gxb, glb, tbuf, sem_g):
    cid = lax.axis_index("c")
    sid = lax.axis_index("s")
    wid = cid * 16 + sid
    lo = wid * RW
    hi = jnp.minimum(lo + RW, M)
    iota = lax.iota(jnp.int32, L)

    pltpu.sync_copy(idx_h, idx_v)
    pltpu.sync_copy(byn_h, byn_v)
    pltpu.sync_copy(tv_h, tbuf)
    tvec = tbuf[...]

    neg1 = jnp.full((L,), -1, jnp.int32)

    def init_b(k, carry):
        win_v[pl.ds(k * L, L)] = neg1
        return carry
    lax.fori_loop(0, RW // L, init_b, 0)

    def scan_b(k, carry):
        v = idx_v[pl.ds(k * L, L)]
        m = (v >= lo) & (v < hi)
        cnt = jnp.sum(m.astype(jnp.int32))
        local = jnp.where(m, v - lo, 0)
        gi = k * L + iota

        @pl.when(cnt == 1)
        def _():
            plsc.store_scatter(win_v, [local], gi, mask=m)

        @pl.when(cnt > 1)
        def _():
            for j in range(L):
                plsc.store_scatter(win_v, [local], gi, mask=m & (iota == j))
        return carry
    lax.fori_loop(0, NCHUNK, scan_b, 0)

    def chunk_b(k, carry):
        r = pl.multiple_of(jnp.minimum(lo + k * CH, M - CH), 8)
        lb = r - lo

        pltpu.sync_copy(bx_h.at[pl.ds(r, CH)], cpx)
        pltpu.sync_copy(lg_h.at[pl.ds(r, CH)], cpl)
        pltpu.sync_copy(by_h.at[pl.ds(r, CH)], cby.at[pl.ds(0, CH)])
        pltpu.sync_copy(bt_h.at[pl.ds(r, CH)], cbt.at[pl.ds(0, CH)])

        def comp_b(j, off):
            w = win_v[pl.ds(lb + j * L, L)]
            mv = w >= 0
            plsc.store_compressed(loc_v.at[pl.ds(off, L)], j * L + iota,
                                  mask=mv)
            plsc.store_compressed(src_v.at[pl.ds(off, L)], w, mask=mv)
            return off + jnp.sum(mv.astype(jnp.int32))
        n_upd = lax.fori_loop(0, LVREG, comp_b, jnp.int32(0))

        def grp_b(g, carry):
            goff = pl.multiple_of(g * L, 8)
            rem = n_upd - g * L
            rvec = loc_v[pl.ds(goff, L)]
            svec = src_v[pl.ds(goff, L)]
            valid = iota < rem
            svec = jnp.where(valid, svec, 0)
            g_x = pltpu.make_async_copy(x_h.at[svec], gxb, sem_g)
            g_l = pltpu.make_async_copy(lnp_h.at[svec], glb, sem_g)
            g_x.start(); g_l.start()
            vals = plsc.load_gather(byn_v, [svec])
            rsafe = jnp.where(valid, rvec, CH)
            plsc.store_scatter(cby, [rsafe], vals, mask=valid)
            plsc.store_scatter(cbt, [rsafe], tvec, mask=valid)
            g_x.wait(); g_l.wait()
            for jl in range(L):
                @pl.when(jl < rem)
                def _():
                    lr = rvec[jl]
                    for cc in range(D // L):
                        cs = pl.ds(cc * L, L)
                        cpx[lr, cs] = gxb[jl, cs]
                    for cc in range(C // L):
                        cs = pl.ds(cc * L, L)
                        cpl[lr, cs] = glb[jl, cs]
                    lrv = iota * 0 + lr
                    tail = glb[jl, pl.ds((C // L) * L, L)]
                    plsc.store_scatter(cpl, [lrv, (C // L) * L + iota],
                                       tail, mask=iota < (C % L))
            return carry
        lax.fori_loop(0, (n_upd + L - 1) // L, grp_b, 0)

        pltpu.sync_copy(cpx, out_bx.at[pl.ds(r, CH)])
        pltpu.sync_copy(cpl, out_lg.at[pl.ds(r, CH)])
        pltpu.sync_copy(cby.at[pl.ds(0, CH)], out_by.at[pl.ds(r, CH)])
        pltpu.sync_copy(cbt.at[pl.ds(0, CH)], out_bt.at[pl.ds(r, CH)])
        return carry
    lax.fori_loop(0, NCPC, chunk_b, 0)


@jax.jit
def _sc_scatter(bx, logits_buf, by_buf, bt_buf, x, lnp, by_new, idx, tvec):
    f = pl.kernel(
        _body,
        out_type=(
            jax.ShapeDtypeStruct((M, D), jnp.float32),
            jax.ShapeDtypeStruct((M,), jnp.int32),
            jax.ShapeDtypeStruct((M,), jnp.int32),
            jax.ShapeDtypeStruct((M, C), jnp.float32),
        ),
        mesh=plsc.VectorSubcoreMesh(core_axis_name="c", subcore_axis_name="s"),
        compiler_params=pltpu.CompilerParams(needs_layout_passes=False),
        scratch_types=[
            pltpu.VMEM((B,), jnp.int32),
            pltpu.VMEM((B,), jnp.int32),
            pltpu.VMEM((RW,), jnp.int32),
            pltpu.VMEM((CAPC,), jnp.int32),
            pltpu.VMEM((CAPC,), jnp.int32),
            pltpu.VMEM((CH, D), jnp.float32),
            pltpu.VMEM((CH, C), jnp.float32),
            pltpu.VMEM((CH + 1,), jnp.int32),
            pltpu.VMEM((CH + 1,), jnp.int32),
            pltpu.VMEM((L, D), jnp.float32),
            pltpu.VMEM((L, D), jnp.float32),
            pltpu.VMEM((L,), jnp.int32),
            pltpu.SemaphoreType.DMA,
        ],
    )
    return f(bx, logits_buf, by_buf, bt_buf, x, lnp, by_new, idx, tvec)


def kernel(bx, logits_buf, by_buf, bt_buf, x, logits_new, by_new, idx, t):
    tvec = jnp.full((L,), t, dtype=jnp.int32)
    lnp = jnp.pad(logits_new, ((0, 0), (0, D - C)))
    return _sc_scatter(bx, logits_buf, by_buf, bt_buf, x, lnp,
                       by_new.astype(jnp.int32), idx.astype(jnp.int32), tvec)
